# Initial kernel scaffold; baseline (speedup 1.0000x reference)
import jax
import jax.numpy as jnp
from jax.experimental import pallas as pl


def kernel(x, edge_index, line_graph_val, W_i, W_j):
    raise NotImplementedError("write your pallas kernel here")


# speedup vs baseline: 14.7534x; 14.7534x over previous
import functools

import jax
import jax.numpy as jnp
from jax import lax
from jax.experimental import pallas as pl
from jax.experimental.pallas import tpu as pltpu
from jax.experimental.pallas import tpu_sc as plsc

N = 10000
E = 320000
H = 128

NC = 2
NS = 16
L = 16

BATCH = 400
ROWS_PER_TILE = E // (BATCH * NS)
ROWS_PER_CORE = ROWS_PER_TILE // NC
NPAD = 10240
WCHUNK = 48
WBASE = 624


def _matvec2_tc(x, w2):
    def body(x_ref, w_ref, o_ref):
        o_ref[...] = jnp.dot(x_ref[...], w_ref[...],
                             preferred_element_type=jnp.float32)

    return pl.pallas_call(
        body,
        out_shape=jax.ShapeDtypeStruct((N, 8), jnp.float32),
    )(x, w2)


def _finish_tc(partials):
    def body(p_ref, o_ref):
        o_ref[...] = jnp.maximum(p_ref[0] + p_ref[1], 0.0)

    grid = 5
    blk = N // grid
    return pl.pallas_call(
        body,
        grid=(grid,),
        in_specs=[pl.BlockSpec((2, blk, H), lambda i: (0, i, 0))],
        out_specs=pl.BlockSpec((blk, H), lambda i: (i, 0)),
        out_shape=jax.ShapeDtypeStruct((N, H), jnp.float32),
    )(partials)


def _gat_sc(x, src4d, dst4d, s_i, s_j):
    mesh = plsc.VectorSubcoreMesh(core_axis_name="c", subcore_axis_name="s")

    @functools.partial(
        pl.kernel,
        out_type=jax.ShapeDtypeStruct((NC, N, H), jnp.float32),
        mesh=mesh,
        compiler_params=pltpu.CompilerParams(needs_layout_passes=False),
        scratch_types=dict(
            si_loc=pltpu.VMEM((N,), jnp.float32),
            sj_loc=pltpu.VMEM((N,), jnp.float32),
            s_loc=pltpu.VMEM((NPAD,), jnp.float32),
            src_row=pltpu.VMEM((BATCH,), jnp.int32),
            dst_row=pltpu.VMEM((BATCH,), jnp.int32),
            t_buf=pltpu.VMEM((BATCH,), jnp.float32),
            rowbuf=pltpu.VMEM((L, H), jnp.float32),
            zblk=pltpu.VMEM((WCHUNK, H), jnp.float32),
            zrow=pltpu.VMEM((NPAD // NS,), jnp.float32),
            s_shared=pltpu.VMEM_SHARED((NPAD,), jnp.float32),
            out_acc=pltpu.VMEM_SHARED((N, H), jnp.float32),
            sem=pltpu.SemaphoreType.DMA,
        ),
    )
    def kern(x_hbm, src_hbm, dst_hbm, si_hbm, sj_hbm, out_hbm, *,
             si_loc, sj_loc, s_loc, src_row, dst_row, t_buf,
             rowbuf, zblk, zrow, s_shared, out_acc, sem):
        c = lax.axis_index("c")
        s = lax.axis_index("s")
        zvec = jnp.zeros((L,), jnp.float32)

        def zero_zblk(i, _):
            for j in range(H // L):
                zblk[i, pl.ds(j * L, L)] = zvec
            return 0
        lax.fori_loop(0, WCHUNK, zero_zblk, 0)
        for i in range(NPAD // NS // L):
            zrow[pl.ds(i * L, L)] = zvec

        pltpu.sync_copy(
            zrow,
            s_shared.at[pl.ds(pl.multiple_of(s * (NPAD // NS), 128),
                              NPAD // NS)])
        for b in range(WBASE // WCHUNK):
            off = pl.multiple_of(s * WBASE + b * WCHUNK, 8)
            pltpu.sync_copy(zblk, out_acc.at[pl.ds(off, WCHUNK)])

        @pl.when(s == NS - 1)
        def _():
            pltpu.sync_copy(zblk.at[pl.ds(0, N - NS * WBASE)],
                            out_acc.at[pl.ds(NS * WBASE, N - NS * WBASE)])

        pltpu.sync_copy(si_hbm, si_loc)
        pltpu.sync_copy(sj_hbm, sj_loc)
        plsc.subcore_barrier()

        def seg_body(b, _):
            pltpu.sync_copy(src_hbm.at[s, b, 0], src_row)
            pltpu.sync_copy(dst_hbm.at[s, b, 0], dst_row)
            for j in range(BATCH // L):
                srcv = src_row[pl.ds(j * L, L)]
                dstv = dst_row[pl.ds(j * L, L)]
                e_i = plsc.load_gather(si_loc, [dstv])
                e_j = plsc.load_gather(sj_loc, [srcv])
                z = e_i + e_j
                e = jnp.where(z >= 0.0, z, 0.01 * z)
                t_buf[pl.ds(j * L, L)] = jnp.exp(e)
            pltpu.sync_copy(t_buf, s_shared.at[src_row], add=True)
            return 0
        lax.fori_loop(0, ROWS_PER_TILE, seg_body, 0)
        plsc.subcore_barrier()

        pltpu.sync_copy(s_shared, s_loc)

        def row_body(r, _):
            b = c * ROWS_PER_CORE + r
            pltpu.sync_copy(src_hbm.at[s, b, 0], src_row)
            pltpu.sync_copy(dst_hbm.at[s, b, 0], dst_row)

            def grp(j, _):
                srcv = src_row[pl.ds(j * L, L)]
                dstv = dst_row[pl.ds(j * L, L)]
                e_i = plsc.load_gather(si_loc, [dstv])
                e_j = plsc.load_gather(sj_loc, [srcv])
                ssum = plsc.load_gather(s_loc, [srcv])
                z = e_i + e_j
                e = jnp.where(z >= 0.0, z, 0.01 * z)
                alpha = jnp.exp(e) / (ssum + 1e-16)
                pltpu.sync_copy(x_hbm.at[srcv], rowbuf)
                for t in range(L):
                    a = alpha[t]
                    for k in range(H // L):
                        rowbuf[t, pl.ds(k * L, L)] = (
                            rowbuf[t, pl.ds(k * L, L)] * a)
                pltpu.sync_copy(rowbuf, out_acc.at[dstv], add=True)
                return 0
            lax.fori_loop(0, BATCH // L, grp, 0)
            return 0
        lax.fori_loop(0, ROWS_PER_CORE, row_body, 0)
        plsc.subcore_barrier()

        for b in range(WBASE // WCHUNK):
            off = pl.multiple_of(s * WBASE + b * WCHUNK, 8)
            pltpu.sync_copy(out_acc.at[pl.ds(off, WCHUNK)],
                            out_hbm.at[c, pl.ds(off, WCHUNK)])

        @pl.when(s == NS - 1)
        def _():
            pltpu.sync_copy(out_acc.at[pl.ds(NS * WBASE, N - NS * WBASE)],
                            out_hbm.at[c, pl.ds(NS * WBASE, N - NS * WBASE)])

    return kern(x, src4d, dst4d, s_i, s_j)


def kernel(x, edge_index, line_graph_val, W_i, W_j):
    del line_graph_val
    w2 = jnp.zeros((H, 8), jnp.float32)
    w2 = w2.at[:, 0].set(W_i).at[:, 1].set(W_j)
    s2 = _matvec2_tc(x, w2)
    s_i = s2[:, 0]
    s_j = s2[:, 1]
    src4d = edge_index[0].reshape(NS, ROWS_PER_TILE, 1, BATCH)
    dst4d = edge_index[1].reshape(NS, ROWS_PER_TILE, 1, BATCH)
    partials = _gat_sc(x, src4d, dst4d, s_i, s_j)
    return _finish_tc(partials)


# speedup vs baseline: 24.4500x; 1.6572x over previous
import functools

import jax
import jax.numpy as jnp
from jax import lax
from jax.experimental import pallas as pl
from jax.experimental.pallas import tpu as pltpu
from jax.experimental.pallas import tpu_sc as plsc

N = 10000
E = 320000
H = 128

NC = 2
NS = 16
L = 16

BATCH = 400
ROWS_PER_TILE = E // (BATCH * NS)
ROWS_PER_CORE = ROWS_PER_TILE // NC
NPAD = 10240
WCHUNK = 24
WBASE = 624
G = 80


def _matvec2_tc(x, w2):
    def body(x_ref, w_ref, o_ref):
        o_ref[...] = jnp.dot(x_ref[...], w_ref[...],
                             preferred_element_type=jnp.float32)

    return pl.pallas_call(
        body,
        out_shape=jax.ShapeDtypeStruct((N, 8), jnp.float32),
    )(x, w2)


def _finish_tc(partials):
    def body(p_ref, o_ref):
        o_ref[...] = jnp.maximum(p_ref[0] + p_ref[1], 0.0)

    grid = 5
    blk = N // grid
    return pl.pallas_call(
        body,
        grid=(grid,),
        in_specs=[pl.BlockSpec((2, blk, H), lambda i: (0, i, 0))],
        out_specs=pl.BlockSpec((blk, H), lambda i: (i, 0)),
        out_shape=jax.ShapeDtypeStruct((N, H), jnp.float32),
    )(partials)


def _gat_sc(x, src4d, dst4d, s_i, s_j):
    mesh = plsc.VectorSubcoreMesh(core_axis_name="c", subcore_axis_name="s")

    @functools.partial(
        pl.kernel,
        out_type=jax.ShapeDtypeStruct((NC, N, H), jnp.float32),
        mesh=mesh,
        compiler_params=pltpu.CompilerParams(needs_layout_passes=False),
        scratch_types=dict(
            si_loc=pltpu.VMEM((N,), jnp.float32),
            sj_loc=pltpu.VMEM((N,), jnp.float32),
            s_loc=pltpu.VMEM((NPAD,), jnp.float32),
            src_row=pltpu.VMEM((BATCH,), jnp.int32),
            dst_row=pltpu.VMEM((BATCH,), jnp.int32),
            t_buf=pltpu.VMEM((BATCH,), jnp.float32),
            bigbuf=pltpu.VMEM((G, H), jnp.float32),
            zblk=pltpu.VMEM((WCHUNK, H), jnp.float32),
            zrow=pltpu.VMEM((NPAD // NS,), jnp.float32),
            s_shared=pltpu.VMEM_SHARED((NPAD,), jnp.float32),
            out_acc=pltpu.VMEM_SHARED((N, H), jnp.float32),
            sem=pltpu.SemaphoreType.DMA,
        ),
    )
    def kern(x_hbm, src_hbm, dst_hbm, si_hbm, sj_hbm, out_hbm, *,
             si_loc, sj_loc, s_loc, src_row, dst_row, t_buf,
             bigbuf, zblk, zrow, s_shared, out_acc, sem):
        c = lax.axis_index("c")
        s = lax.axis_index("s")
        zvec = jnp.zeros((L,), jnp.float32)

        def zero_zblk(i, _):
            for j in range(H // L):
                zblk[i, pl.ds(j * L, L)] = zvec
            return 0
        lax.fori_loop(0, WCHUNK, zero_zblk, 0)
        for i in range(NPAD // NS // L):
            zrow[pl.ds(i * L, L)] = zvec

        pltpu.sync_copy(
            zrow,
            s_shared.at[pl.ds(pl.multiple_of(s * (NPAD // NS), 128),
                              NPAD // NS)])
        for b in range(WBASE // WCHUNK):
            off = pl.multiple_of(s * WBASE + b * WCHUNK, 8)
            pltpu.sync_copy(zblk, out_acc.at[pl.ds(off, WCHUNK)])

        @pl.when(s == NS - 1)
        def _():
            pltpu.sync_copy(zblk.at[pl.ds(0, N - NS * WBASE)],
                            out_acc.at[pl.ds(NS * WBASE, N - NS * WBASE)])

        pltpu.sync_copy(si_hbm, si_loc)
        pltpu.sync_copy(sj_hbm, sj_loc)
        plsc.subcore_barrier()

        def seg_body(b, _):
            pltpu.sync_copy(src_hbm.at[s, b, 0], src_row)
            pltpu.sync_copy(dst_hbm.at[s, b, 0], dst_row)
            for j in range(BATCH // L):
                srcv = src_row[pl.ds(j * L, L)]
                dstv = dst_row[pl.ds(j * L, L)]
                e_i = plsc.load_gather(si_loc, [dstv])
                e_j = plsc.load_gather(sj_loc, [srcv])
                z = e_i + e_j
                e = jnp.where(z >= 0.0, z, 0.01 * z)
                t_buf[pl.ds(j * L, L)] = jnp.exp(e)
            pltpu.sync_copy(t_buf, s_shared.at[src_row], add=True)
            return 0
        lax.fori_loop(0, ROWS_PER_TILE, seg_body, 0)
        plsc.subcore_barrier()

        pltpu.sync_copy(s_shared, s_loc)

        def row_body(r, _):
            b = c * ROWS_PER_CORE + r
            pltpu.sync_copy(src_hbm.at[s, b, 0], src_row)
            pltpu.sync_copy(dst_hbm.at[s, b, 0], dst_row)

            def grp(j, _):
                off = pl.multiple_of(j * G, 8)
                pltpu.async_copy(
                    x_hbm.at[src_row.at[pl.ds(off, G)]], bigbuf, sem).wait()
                for v in range(G // L):
                    srcv = src_row[pl.ds(off + v * L, L)]
                    dstv = dst_row[pl.ds(off + v * L, L)]
                    e_i = plsc.load_gather(si_loc, [dstv])
                    e_j = plsc.load_gather(sj_loc, [srcv])
                    ssum = plsc.load_gather(s_loc, [srcv])
                    z = e_i + e_j
                    e = jnp.where(z >= 0.0, z, 0.01 * z)
                    alpha = jnp.exp(e) / (ssum + 1e-16)
                    for t in range(L):
                        a = alpha[t]
                        r_i = v * L + t
                        for k in range(H // L):
                            bigbuf[r_i, pl.ds(k * L, L)] = (
                                bigbuf[r_i, pl.ds(k * L, L)] * a)
                pltpu.sync_copy(bigbuf,
                                out_acc.at[dst_row.at[pl.ds(off, G)]],
                                add=True)
                return 0
            lax.fori_loop(0, BATCH // G, grp, 0)
            return 0
        lax.fori_loop(0, ROWS_PER_CORE, row_body, 0)
        plsc.subcore_barrier()

        for b in range(WBASE // WCHUNK):
            off = pl.multiple_of(s * WBASE + b * WCHUNK, 8)
            pltpu.sync_copy(out_acc.at[pl.ds(off, WCHUNK)],
                            out_hbm.at[c, pl.ds(off, WCHUNK)])

        @pl.when(s == NS - 1)
        def _():
            pltpu.sync_copy(out_acc.at[pl.ds(NS * WBASE, N - NS * WBASE)],
                            out_hbm.at[c, pl.ds(NS * WBASE, N - NS * WBASE)])

    return kern(x, src4d, dst4d, s_i, s_j)


def kernel(x, edge_index, line_graph_val, W_i, W_j):
    del line_graph_val
    w2 = jnp.zeros((H, 8), jnp.float32)
    w2 = w2.at[:, 0].set(W_i).at[:, 1].set(W_j)
    s2 = _matvec2_tc(x, w2)
    s_i = s2[:, 0]
    s_j = s2[:, 1]
    src4d = edge_index[0].reshape(NS, ROWS_PER_TILE, 1, BATCH)
    dst4d = edge_index[1].reshape(NS, ROWS_PER_TILE, 1, BATCH)
    partials = _gat_sc(x, src4d, dst4d, s_i, s_j)
    return _finish_tc(partials)


# speedup vs baseline: 36.4661x; 1.4915x over previous
import functools

import jax
import jax.numpy as jnp
from jax import lax
from jax.experimental import pallas as pl
from jax.experimental.pallas import tpu as pltpu
from jax.experimental.pallas import tpu_sc as plsc

N = 10000
E = 320000
H = 128

NC = 2
NS = 16
L = 16

BATCH = 400
PB = 2 * BATCH
ROWS_PER_TILE = E // (BATCH * NS)
ROWS_PER_CORE = ROWS_PER_TILE // NC
EDGES_P2 = ROWS_PER_CORE * BATCH
NPAD = 10240
WBASE = 624
G = 80
GPR = BATCH // G
NG = EDGES_P2 // G


def _matvec2_tc(x, w2):
    def body(x_ref, w_ref, o_ref):
        o_ref[...] = jnp.dot(x_ref[...], w_ref[...],
                             preferred_element_type=jnp.float32)

    return pl.pallas_call(
        body,
        out_shape=jax.ShapeDtypeStruct((N, 8), jnp.float32),
    )(x, w2)


def _finish_tc(partials):
    def body(p_ref, o_ref):
        o_ref[...] = jnp.maximum(p_ref[0] + p_ref[1], 0.0)

    grid = 5
    blk = N // grid
    return pl.pallas_call(
        body,
        grid=(grid,),
        in_specs=[pl.BlockSpec((2, blk, H), lambda i: (0, i, 0))],
        out_specs=pl.BlockSpec((blk, H), lambda i: (i, 0)),
        out_shape=jax.ShapeDtypeStruct((N, H), jnp.float32),
    )(partials)


def _gat_sc(x, packed, s_i, s_j):
    mesh = plsc.VectorSubcoreMesh(core_axis_name="c", subcore_axis_name="s")

    @functools.partial(
        pl.kernel,
        out_type=jax.ShapeDtypeStruct((NC, N, H), jnp.float32),
        mesh=mesh,
        compiler_params=pltpu.CompilerParams(needs_layout_passes=False),
        scratch_types=dict(
            s_loc=pltpu.VMEM((NPAD,), jnp.float32),
            t_mine=pltpu.VMEM((EDGES_P2,), jnp.float32),
            er0=pltpu.VMEM((PB,), jnp.int32),
            er1=pltpu.VMEM((PB,), jnp.int32),
            er2=pltpu.VMEM((PB,), jnp.int32),
            tb0=pltpu.VMEM((BATCH,), jnp.float32),
            tb1=pltpu.VMEM((BATCH,), jnp.float32),
            zrow=pltpu.VMEM((NPAD // NS,), jnp.float32),
            s_shared=pltpu.VMEM_SHARED((NPAD,), jnp.float32),
            out_acc=pltpu.VMEM_SHARED((N, H), jnp.float32),
            stsem=pltpu.SemaphoreType.DMA,
            scsem0=pltpu.SemaphoreType.DMA,
            scsem1=pltpu.SemaphoreType.DMA,
            gsem0=pltpu.SemaphoreType.DMA,
            gsem1=pltpu.SemaphoreType.DMA,
            ssem0=pltpu.SemaphoreType.DMA,
            ssem1=pltpu.SemaphoreType.DMA,
        ),
    )
    def kern(x_hbm, ep_hbm, si_hbm, sj_hbm, out_hbm, *,
             s_loc, t_mine, er0, er1, er2, tb0, tb1, zrow,
             s_shared, out_acc, stsem, scsem0, scsem1,
             gsem0, gsem1, ssem0, ssem1):
        c = lax.axis_index("c")
        s = lax.axis_index("s")
        zvec = jnp.zeros((L,), jnp.float32)
        ers = [er0, er1, er2]
        tbs = [tb0, tb1]
        scs = [scsem0, scsem1]

        for i in range(NPAD // NS // L):
            zrow[pl.ds(i * L, L)] = zvec
        pltpu.sync_copy(
            zrow,
            s_shared.at[pl.ds(pl.multiple_of(s * (NPAD // NS), 128),
                              NPAD // NS)])

        def phase1(si_loc, sj_loc):
            pltpu.sync_copy(si_hbm, si_loc)
            pltpu.sync_copy(sj_hbm, sj_loc)
            plsc.subcore_barrier()
            pltpu.sync_copy(ep_hbm.at[s, 0, 0], er0)

            def seg_step(b, k):
                er_cur = ers[k % 3]
                er_nxt = ers[(k + 1) % 3]
                tb_cur = tbs[k % 2]
                sc_cur = scs[k % 2]

                @pl.when(b > 1)
                def _():
                    pltpu.make_async_copy(
                        tb_cur, s_shared.at[er_nxt.at[pl.ds(0, BATCH)]],
                        sc_cur).wait()

                @pl.when(b < ROWS_PER_TILE - 1)
                def _():
                    pltpu.async_copy(ep_hbm.at[s, b + 1, 0], er_nxt, stsem)

                base = c * ROWS_PER_CORE
                for j in range(BATCH // L):
                    srcv = er_cur[pl.ds(j * L, L)]
                    dstv = er_cur[pl.ds(BATCH + j * L, L)]
                    e_i = plsc.load_gather(si_loc, [dstv])
                    e_j = plsc.load_gather(sj_loc, [srcv])
                    z = e_i + e_j
                    e = jnp.where(z >= 0.0, z, 0.01 * z)
                    t = jnp.exp(e)
                    tb_cur[pl.ds(j * L, L)] = t

                    @pl.when(jnp.logical_and(b >= base,
                                             b < base + ROWS_PER_CORE))
                    def _():
                        t_mine[pl.ds((b - base) * BATCH + j * L, L)] = t

                pltpu.async_copy(
                    tb_cur, s_shared.at[er_cur.at[pl.ds(0, BATCH)]],
                    sc_cur, add=True)

                @pl.when(b < ROWS_PER_TILE - 1)
                def _():
                    pltpu.make_async_copy(ep_hbm.at[s, 0, 0], er_nxt,
                                          stsem).wait()

            def seg_body(b, _):
                for k in range(6):
                    @pl.when(lax.rem(b, 6) == k)
                    def _(k=k):
                        seg_step(b, k)
                return 0
            lax.fori_loop(0, ROWS_PER_TILE, seg_body, 0)
            pltpu.make_async_copy(
                tb0, s_shared.at[er0.at[pl.ds(0, BATCH)]], scsem0).wait()
            pltpu.make_async_copy(
                tb1, s_shared.at[er0.at[pl.ds(0, BATCH)]], scsem1).wait()

        pl.run_scoped(phase1,
                      pltpu.VMEM((N,), jnp.float32),
                      pltpu.VMEM((N,), jnp.float32))
        plsc.subcore_barrier()
        pltpu.sync_copy(s_shared, s_loc)

        def phase2(b0, b1):
            def zero_b0(i, _):
                for j in range(H // L):
                    b0[i, pl.ds(j * L, L)] = zvec
                return 0
            lax.fori_loop(0, G, zero_b0, 0)
            for q in range(WBASE // G):
                off = pl.multiple_of(s * WBASE + q * G, 8)
                pltpu.sync_copy(b0, out_acc.at[pl.ds(off, G)])
            offr = pl.multiple_of(s * WBASE + (WBASE // G) * G, 8)
            pltpu.sync_copy(b0.at[pl.ds(0, WBASE - (WBASE // G) * G)],
                            out_acc.at[pl.ds(offr,
                                             WBASE - (WBASE // G) * G)])

            @pl.when(s == NS - 1)
            def _():
                pltpu.sync_copy(b0.at[pl.ds(0, N - NS * WBASE)],
                                out_acc.at[pl.ds(NS * WBASE,
                                                 N - NS * WBASE)])
            plsc.subcore_barrier()

            pltpu.sync_copy(ep_hbm.at[s, c * ROWS_PER_CORE, 0], er0)
            pltpu.async_copy(x_hbm.at[er0.at[pl.ds(0, G)]], b0, gsem0)

            def scale(g, X):
                jr = lax.rem(g, GPR)
                for v in range(G // L):
                    tv = t_mine[pl.ds(g * G + v * L, L)]
                    srcv = er0[pl.ds(jr * G + v * L, L)]
                    ssum = plsc.load_gather(s_loc, [srcv])
                    alpha = tv / (ssum + 1e-16)
                    for t in range(L):
                        a = alpha[t]
                        r_i = v * L + t
                        for k in range(H // L):
                            X[r_i, pl.ds(k * L, L)] = (
                                X[r_i, pl.ds(k * L, L)] * a)

            def grp_step(g, X, gsX, ssX, Y, gsY, ssY):
                jr = lax.rem(g, GPR)
                pltpu.make_async_copy(x_hbm.at[er0.at[pl.ds(0, G)]], X,
                                      gsX).wait()

                @pl.when(jnp.logical_and(g > 0, jr != 0))
                def _():
                    pltpu.make_async_copy(
                        Y, out_acc.at[er0.at[pl.ds(0, G)]], ssY).wait()

                @pl.when(jnp.logical_and(jr != GPR - 1, g < NG - 1))
                def _():
                    off = pl.multiple_of((jr + 1) * G, 8)
                    pltpu.async_copy(x_hbm.at[er0.at[pl.ds(off, G)]], Y,
                                     gsY)

                scale(g, X)
                off_d = pl.multiple_of(BATCH + jr * G, 8)
                pltpu.async_copy(X, out_acc.at[er0.at[pl.ds(off_d, G)]],
                                 ssX, add=True)

                @pl.when(jnp.logical_and(jr == GPR - 1, g < NG - 1))
                def _():
                    pltpu.make_async_copy(
                        X, out_acc.at[er0.at[pl.ds(0, G)]], ssX).wait()
                    r = lax.div(g, GPR)
                    pltpu.sync_copy(
                        ep_hbm.at[s, c * ROWS_PER_CORE + r + 1, 0], er0)
                    pltpu.async_copy(x_hbm.at[er0.at[pl.ds(0, G)]], Y,
                                     gsY)

            def grp_body(g, _):
                @pl.when((g & 1) == 0)
                def _():
                    grp_step(g, b0, gsem0, ssem0, b1, gsem1, ssem1)

                @pl.when((g & 1) == 1)
                def _():
                    grp_step(g, b1, gsem1, ssem1, b0, gsem0, ssem0)
                return 0
            lax.fori_loop(0, NG, grp_body, 0)
            pltpu.make_async_copy(
                b0, out_acc.at[er0.at[pl.ds(0, G)]], ssem0).wait()

        pl.run_scoped(phase2,
                      pltpu.VMEM((G, H), jnp.float32),
                      pltpu.VMEM((G, H), jnp.float32))
        plsc.subcore_barrier()

        off_w = pl.multiple_of(s * WBASE, 8)
        pltpu.sync_copy(out_acc.at[pl.ds(off_w, WBASE)],
                        out_hbm.at[c, pl.ds(off_w, WBASE)])

        @pl.when(s == NS - 1)
        def _():
            pltpu.sync_copy(out_acc.at[pl.ds(NS * WBASE, N - NS * WBASE)],
                            out_hbm.at[c, pl.ds(NS * WBASE,
                                                N - NS * WBASE)])

    return kern(x, packed, s_i, s_j)


def kernel(x, edge_index, line_graph_val, W_i, W_j):
    del line_graph_val
    w2 = jnp.zeros((H, 8), jnp.float32)
    w2 = w2.at[:, 0].set(W_i).at[:, 1].set(W_j)
    s2 = _matvec2_tc(x, w2)
    s_i = s2[:, 0]
    s_j = s2[:, 1]
    src = edge_index[0].reshape(NS, ROWS_PER_TILE, 1, BATCH)
    dst = edge_index[1].reshape(NS, ROWS_PER_TILE, 1, BATCH)
    packed = jnp.concatenate([src, dst], axis=3)
    partials = _gat_sc(x, packed, s_i, s_j)
    return _finish_tc(partials)
